# baseline (speedup 1.0000x reference)
import functools

import jax
import jax.numpy as jnp
from jax import lax
from jax.experimental import pallas as pl
from jax.experimental.pallas import tpu as pltpu
from jax.experimental.pallas import tpu_sc as plsc

S, T = 4096, 200
D = 64
NC, NS = 2, 16
NW = NC * NS
SPW = S // NW
CS = 2
NCHUNK = SPW // CS
NBUF = 2

_vector_mesh = plsc.VectorSubcoreMesh(
    core_axis_name="core", subcore_axis_name="subcore"
)


@jax.jit
def _gather_sc(table, indices):
    @functools.partial(
        pl.kernel,
        out_type=jax.ShapeDtypeStruct((S, T, D), jnp.float32),
        mesh=_vector_mesh,
        scratch_types=[
            pltpu.VMEM((NBUF, CS, T), jnp.int32),
            pltpu.VMEM((NBUF, CS, T, D), jnp.float32),
            pltpu.SemaphoreType.DMA((NBUF,)),
            pltpu.SemaphoreType.DMA((NBUF,)),
            pltpu.SemaphoreType.DMA((NBUF,)),
        ],
        compiler_params=pltpu.CompilerParams(use_tc_tiling_on_sc=False),
    )
    def kern(tab_hbm, idx_hbm, out_hbm, idx_v, rows_v, isem, gsem, osem):
        wid = lax.axis_index("subcore") * NC + lax.axis_index("core")
        base = wid * SPW

        for b in range(NBUF):
            pltpu.async_copy(
                idx_hbm.at[pl.ds(base + b * CS, CS)], idx_v.at[b], isem.at[b]
            )

        @pl.loop(0, NCHUNK, step=NBUF)
        def _(i):
            for b in range(NBUF):
                s0 = base + (i + b) * CS

                @pl.when(i > 0)
                def _():
                    pltpu.make_async_copy(
                        rows_v.at[b], out_hbm.at[pl.ds(s0, CS)], osem.at[b]
                    ).wait()

                pltpu.make_async_copy(
                    idx_hbm.at[pl.ds(s0, CS)], idx_v.at[b], isem.at[b]
                ).wait()

                for j in range(CS):
                    pltpu.async_copy(
                        tab_hbm.at[idx_v.at[b, j]],
                        rows_v.at[b, j],
                        gsem.at[b],
                    )
                for j in range(CS):
                    pltpu.make_async_copy(
                        tab_hbm.at[idx_v.at[b, j]],
                        rows_v.at[b, j],
                        gsem.at[b],
                    ).wait()

                @pl.when(i + NBUF < NCHUNK)
                def _():
                    pltpu.async_copy(
                        idx_hbm.at[pl.ds(s0 + NBUF * CS, CS)],
                        idx_v.at[b],
                        isem.at[b],
                    )

                pltpu.async_copy(
                    rows_v.at[b], out_hbm.at[pl.ds(s0, CS)], osem.at[b]
                )

        for b in range(NBUF):
            pltpu.make_async_copy(
                rows_v.at[b], out_hbm.at[pl.ds(base, CS)], osem.at[b]
            ).wait()

    return kern(table, indices)


def kernel(tokenized_sentence, table):
    return _gather_sc(table, tokenized_sentence)
